# baseline (speedup 1.0000x reference)
import functools
import jax
import jax.numpy as jnp
from jax import lax
from jax.experimental import pallas as pl
from jax.experimental.pallas import tpu as pltpu
from jax.experimental.pallas import tpu_sc as plsc

B = 4096
H = 20
D = 300
V = 100000
L = 16
NC = 2
NS = 16
NW = NC * NS
ROWS_PER_W = B // NW
G = 4
NG = ROWS_PER_W // G
GH = G * H
NCHUNK = (D + L - 1) // L
TAIL0 = D - 128

_mesh = plsc.VectorSubcoreMesh(core_axis_name="c", subcore_axis_name="s")


@functools.partial(
    pl.kernel,
    mesh=_mesh,
    out_type=jax.ShapeDtypeStruct((B, D), jnp.float32),
    scratch_types=[
        pltpu.VMEM((2, GH), jnp.int32),
        pltpu.VMEM((2, GH), jnp.float32),
        pltpu.VMEM((2, GH, 128), jnp.float32),
        pltpu.VMEM((2, GH, 128), jnp.float32),
        pltpu.VMEM((2, GH, 128), jnp.float32),
        pltpu.VMEM((2, G, D), jnp.float32),
        pltpu.SemaphoreType.DMA,
        pltpu.SemaphoreType.DMA,
        pltpu.SemaphoreType.DMA,
    ],
)
def _embed_sum(idx_hbm, w_hbm, table_hbm, tail_hbm, out_hbm,
               idx_v, w_v, ra_v, rb_v, rc_v, out_v, sem0, sem1, osem):
    wid = lax.axis_index("s") * NC + lax.axis_index("c")
    g0 = wid * NG
    base0 = wid * ROWS_PER_W
    sems = [sem0, sem1]

    def load_iw(g, p):
        pltpu.sync_copy(idx_hbm.at[pl.ds(g * GH, GH)], idx_v.at[p])
        pltpu.sync_copy(w_hbm.at[pl.ds(g * GH, GH)], w_v.at[p])

    def fire(p):
        idx = idx_v.at[p]
        pltpu.async_copy(table_hbm.at[idx, pl.ds(0, 128)], ra_v.at[p], sems[p])
        pltpu.async_copy(table_hbm.at[idx, pl.ds(128, 128)], rb_v.at[p], sems[p])
        pltpu.async_copy(tail_hbm.at[idx], rc_v.at[p], sems[p])

    def chunk(p, k, c):
        off = min(c * L, D - L)
        if off < 128:
            return ra_v[p, k, pl.ds(off, L)]
        if off < 256:
            return rb_v[p, k, pl.ds(off - 128, L)]
        return rc_v[p, k, pl.ds(off - TAIL0, L)]

    def compute(g, p):
        for buf in (ra_v, rb_v, rc_v):
            pltpu.make_async_copy(tail_hbm.at[pl.ds(0, GH)], buf.at[p],
                                  sems[p]).wait()
        for r in range(G):
            wa = w_v[p, pl.ds(r * H, L)]
            wb = w_v[p, pl.ds(r * H + H - L, L)]
            wvs = [wa[h] if h < L else wb[h - (H - L)] for h in range(H)]
            for c in range(NCHUNK):
                off = min(c * L, D - L)
                acc = wvs[0] * chunk(p, r * H, c)
                for h in range(1, H):
                    acc = acc + wvs[h] * chunk(p, r * H + h, c)
                out_v[p, r, pl.ds(off, L)] = acc
        base = base0 + (g - g0) * G
        pltpu.async_copy(out_v.at[p], out_hbm.at[pl.ds(base, G)], osem)

    load_iw(g0, 0)
    fire(0)
    load_iw(g0 + 1, 1)

    def body(g2, carry):
        for p in (0, 1):
            g = g0 + 2 * g2 + p
            cur, nxt = p, 1 - p

            @pl.when(g - g0 < NG - 1)
            def _():
                fire(nxt)

            @pl.when(g - g0 >= 2)
            def _():
                pltpu.make_async_copy(
                    out_v.at[cur], out_hbm.at[pl.ds(base0, G)], osem).wait()

            compute(g, cur)

            @pl.when(g - g0 < NG - 2)
            def _():
                load_iw(g + 2, cur)
        return carry

    lax.fori_loop(0, NG // 2, body, 0)
    for p in (0, 1):
        pltpu.make_async_copy(out_v.at[p], out_hbm.at[pl.ds(base0, G)],
                              osem).wait()


def kernel(batch_hnym, batch_hnym_weights, table):
    idx = batch_hnym.reshape(-1).astype(jnp.int32)
    w = batch_hnym_weights.reshape(-1)
    tail = table[:, TAIL0:]
    return _embed_sum(idx, w, table, tail)
